# baseline (speedup 1.0000x reference)
import functools

import jax
import jax.numpy as jnp
from jax import lax
from jax.experimental import pallas as pl
from jax.experimental.pallas import tpu as pltpu
from jax.experimental.pallas import tpu_sc as plsc

B = 4096
H = 50
D = 64
TOTAL = B * H
NC, NS = 2, 16
NW = NC * NS
PER_W = TOTAL // NW
CHUNK = 128
NCHUNK = PER_W // CHUNK
NBUF = 5
LOOK = 2

_mesh = plsc.VectorSubcoreMesh(core_axis_name="c", subcore_axis_name="s")


@functools.partial(
    pl.kernel,
    mesh=_mesh,
    out_type=jax.ShapeDtypeStruct((TOTAL // CHUNK, CHUNK, 2 * D), jnp.float32),
    scratch_types=[
        pltpu.VMEM((PER_W,), jnp.int32),
        pltpu.VMEM((NBUF, CHUNK, 2 * D), jnp.float32),
        [pltpu.SemaphoreType.DMA] * NBUF,
        [pltpu.SemaphoreType.DMA] * NBUF,
    ],
    compiler_params=pltpu.CompilerParams(use_tc_tiling_on_sc=True),
)
def _emb_lookup(idx_hbm, table_hbm, out_hbm, idx_v, rows_v, gsems, wsems):
    wid = lax.axis_index("s") * NC + lax.axis_index("c")
    blk0 = wid * NCHUNK

    pltpu.sync_copy(idx_hbm.at[wid], idx_v)

    def fire_gathers(c, b):
        pltpu.async_copy(table_hbm.at[idx_v.at[pl.ds(c * CHUNK, CHUNK)]],
                         rows_v.at[b], gsems[b])

    def wait_gathers(c, b):
        pltpu.make_async_copy(table_hbm.at[idx_v.at[pl.ds(c * CHUNK, CHUNK)]],
                              rows_v.at[b], gsems[b]).wait()

    def fire_writeback(c, b):
        pltpu.async_copy(rows_v.at[b], out_hbm.at[blk0 + c], wsems[b])

    def wait_writeback(c, b):
        pltpu.make_async_copy(rows_v.at[b], out_hbm.at[blk0 + c],
                              wsems[b]).wait()

    for b in range(LOOK):
        fire_gathers(b, b)

    def step(c, b):
        wait_gathers(c, b)
        fire_writeback(c, b)
        n = c + LOOK
        bn = (b + LOOK) % NBUF

        @pl.when(n < NCHUNK)
        def _():
            @pl.when(n >= NBUF)
            def _():
                wait_writeback(n - NBUF, bn)

            fire_gathers(n, bn)
        return 0

    lax.fori_loop(
        0, NCHUNK // NBUF,
        lambda i, x: [step(i * NBUF + b, b) for b in range(NBUF)][-1],
        0, unroll=False)

    for m in range(NCHUNK - NBUF, NCHUNK):
        wait_writeback(m, m % NBUF)


def kernel(token_ids, weight):
    wpad = jnp.pad(weight, ((0, 0), (0, D)))
    idx = token_ids.reshape(NW, PER_W).astype(jnp.int32)
    out = _emb_lookup(idx, wpad)
    return out.reshape(B, H, 2 * D)[:, :, :D]


# speedup vs baseline: 1.1090x; 1.1090x over previous
import functools

import jax
import jax.numpy as jnp
from jax import lax
from jax.experimental import pallas as pl
from jax.experimental.pallas import tpu as pltpu
from jax.experimental.pallas import tpu_sc as plsc

B = 4096
H = 50
D = 64
NC, NS = 2, 16
NW = NC * NS
TPW = B // NW
TPC = 2
NCHUNK = TPW // TPC
NBUF = 2
LOOK = 1
L = 16

_mesh = plsc.VectorSubcoreMesh(core_axis_name="c", subcore_axis_name="s")


@functools.partial(
    pl.kernel,
    mesh=_mesh,
    out_type=jax.ShapeDtypeStruct((B, H, D), jnp.float32),
    scratch_types=[
        pltpu.VMEM((TPW, H), jnp.int32),
        pltpu.VMEM((NBUF, TPC, H, 2 * D), jnp.float32),
        pltpu.VMEM((NBUF, TPC, H, D), jnp.float32),
        [pltpu.SemaphoreType.DMA] * NBUF,
        [pltpu.SemaphoreType.DMA] * NBUF,
    ],
    compiler_params=pltpu.CompilerParams(use_tc_tiling_on_sc=True),
)
def _emb_lookup(idx_hbm, table_hbm, out_hbm, idx_v, gbuf, tbuf, gsems, wsems):
    wid = lax.axis_index("s") * NC + lax.axis_index("c")
    tok0 = wid * TPW

    pltpu.sync_copy(idx_hbm.at[wid], idx_v)

    def fire_gathers(c, b):
        for j in range(TPC):
            pltpu.async_copy(table_hbm.at[idx_v.at[c * TPC + j]],
                             gbuf.at[b, j], gsems[b])

    def wait_gathers(c, b):
        for j in range(TPC):
            pltpu.make_async_copy(table_hbm.at[idx_v.at[c * TPC + j]],
                                  gbuf.at[b, j], gsems[b]).wait()

    def relayout(b):
        def row(h, carry):
            for j in range(TPC):
                for q in range(D // L):
                    tbuf[b, j, h, pl.ds(q * L, L)] = (
                        gbuf[b, j, h, pl.ds(q * L, L)])
            return carry
        lax.fori_loop(0, H, row, 0, unroll=False)

    def fire_writeback(c, b):
        pltpu.async_copy(tbuf.at[b],
                         out_hbm.at[pl.ds(tok0 + c * TPC, TPC)], wsems[b])

    def wait_writeback(c, b):
        pltpu.make_async_copy(tbuf.at[b],
                              out_hbm.at[pl.ds(tok0 + c * TPC, TPC)],
                              wsems[b]).wait()

    for b in range(LOOK):
        fire_gathers(b, b)

    def step(c, b):
        wait_gathers(c, b)
        relayout(b)
        fire_writeback(c, b)
        n = c + LOOK
        bn = (b + LOOK) % NBUF

        @pl.when(n < NCHUNK)
        def _():
            @pl.when(n >= NBUF)
            def _():
                wait_writeback(n - NBUF, bn)

            fire_gathers(n, bn)
        return 0

    lax.fori_loop(
        0, NCHUNK // NBUF,
        lambda i, x: [step(i * NBUF + b, b) for b in range(NBUF)][-1],
        0, unroll=False)

    for m in range(NCHUNK - NBUF, NCHUNK):
        wait_writeback(m, m % NBUF)


def kernel(token_ids, weight):
    wpad = jnp.pad(weight, ((0, 0), (0, D)))
    idx = token_ids.reshape(NW, TPW, H).astype(jnp.int32)
    return _emb_lookup(idx, wpad)
